# baseline (speedup 1.0000x reference)
import functools

import jax
import jax.numpy as jnp
from jax import lax
from jax.experimental import pallas as pl
from jax.experimental.pallas import tpu as pltpu
from jax.experimental.pallas import tpu_sc as plsc

BATCH = 16384
NUM_USERS = 1000000
NUM_MOVIES = 100000
UD = 32
MD = 32
H1 = 128
H2 = 64
GRP = 128

NUM_WORKERS = 32
CHUNK = 128
ROWS_PER_W = BATCH // NUM_WORKERS
CHUNKS_PER_W = ROWS_PER_W // CHUNK
IDX_ROWS = BATCH // CHUNK


def _sc_gather(t4, idx2d):
    mesh = plsc.VectorSubcoreMesh(core_axis_name="c", subcore_axis_name="s")

    @functools.partial(
        pl.kernel,
        mesh=mesh,
        compiler_params=pltpu.CompilerParams(use_tc_tiling_on_sc=False),
        out_type=jax.ShapeDtypeStruct((BATCH, GRP), jnp.float32),
        scratch_types=[
            pltpu.VMEM((CHUNKS_PER_W, CHUNK), jnp.int32),
            pltpu.VMEM((ROWS_PER_W, GRP), jnp.float32),
            pltpu.SemaphoreType.DMA,
        ],
    )
    def k(t_hbm, i_hbm, out_hbm, i_v, rows_v, sem):
        wid = lax.axis_index("s") * 2 + lax.axis_index("c")
        pltpu.sync_copy(i_hbm.at[pl.ds(wid * CHUNKS_PER_W, CHUNKS_PER_W)], i_v)
        copies = []
        for j in range(CHUNKS_PER_W):
            copies.append(pltpu.async_copy(
                t_hbm.at[i_v.at[j]],
                rows_v.at[pl.ds(j * CHUNK, CHUNK)], sem))
        for c in copies:
            c.wait()
        pltpu.sync_copy(rows_v, out_hbm.at[pl.ds(wid * ROWS_PER_W, ROWS_PER_W)])

    return k(t4, idx2d)


RSUB = 12544
DU = 250880
DM = 25088


def _regroup_body(x0_ref, x1_ref, x2_ref, x3_ref, o_ref):
    eye = jnp.eye(UD, dtype=jnp.float32)
    tr = lambda x: lax.dot_general(
        x, eye, (((0,), (0,)), ((), ())),
        preferred_element_type=jnp.float32)
    o_ref[:, 0:UD] = tr(x0_ref[...])
    o_ref[:, UD:2 * UD] = tr(x1_ref[...])
    o_ref[:, 2 * UD:3 * UD] = tr(x2_ref[...])
    o_ref[:, 3 * UD:4 * UD] = tr(x3_ref[...])


def _tc_regroup(tT, d):
    grid = d // RSUB
    specs = [
        pl.BlockSpec((UD, RSUB), lambda i, a=a, g=grid: (0, g * a + i))
        for a in range(4)
    ]
    return pl.pallas_call(
        _regroup_body,
        grid=(grid,),
        in_specs=specs,
        out_specs=pl.BlockSpec((RSUB, GRP), lambda i: (i, 0)),
        out_shape=jax.ShapeDtypeStruct((d, GRP), jnp.float32),
        compiler_params=pltpu.CompilerParams(
            dimension_semantics=("arbitrary",),
            vmem_limit_bytes=56 * 1024 * 1024),
    )(tT, tT, tT, tT)


BLK = 2048
GRID = BATCH // BLK


def _mlp_body(ug_ref, mg_ref, aux_ref, w1u_ref, w1m_ref, war_ref, b1_ref,
              w2_ref, b2_ref, w3_ref, b3_ref, out_ref):
    hp = jax.lax.Precision.DEFAULT
    f32 = jnp.float32
    aux = aux_ref[...]
    auxc = jnp.transpose(aux)
    uoff = auxc[:, 0:1].astype(jnp.int32)
    moff = auxc[:, 1:2].astype(jnp.int32)
    age = auxc[:, 2:3]
    rat = auxc[:, 3:4]
    colgrp = lax.broadcasted_iota(jnp.int32, (BLK, GRP), 1) // UD
    um = jnp.where(colgrp == uoff, ug_ref[...], 0.0)
    mm = jnp.where(colgrp == moff, mg_ref[...], 0.0)
    h = jnp.dot(um, w1u_ref[...], precision=hp, preferred_element_type=f32)
    h = h + jnp.dot(mm, w1m_ref[...], precision=hp, preferred_element_type=f32)
    war = war_ref[...]
    h = h + age * war[0:1, :] + rat * war[1:2, :]
    h = jnp.maximum(h + b1_ref[...], 0.0)
    h = jnp.dot(h, w2_ref[...], precision=hp, preferred_element_type=f32)
    h = jnp.maximum(h + b2_ref[...], 0.0)
    logit = jnp.sum(h * w3_ref[...], axis=1, keepdims=True) + b3_ref[...]
    out_ref[...] = jax.nn.sigmoid(logit)


def _mlp(ug, mg, aux, W1u4, W1m4, War, b1, W2, b2, w3row, b3):
    full = lambda i: (0, 0)
    out = pl.pallas_call(
        _mlp_body,
        grid=(GRID,),
        in_specs=[
            pl.BlockSpec((BLK, GRP), lambda i: (i, 0)),
            pl.BlockSpec((BLK, GRP), lambda i: (i, 0)),
            pl.BlockSpec((4, BLK), lambda i: (0, i)),
            pl.BlockSpec((GRP, H1), full),
            pl.BlockSpec((GRP, H1), full),
            pl.BlockSpec((2, H1), full),
            pl.BlockSpec((1, H1), full),
            pl.BlockSpec((H1, H2), full),
            pl.BlockSpec((1, H2), full),
            pl.BlockSpec((1, H2), full),
            pl.BlockSpec((1, 1), full),
        ],
        out_specs=pl.BlockSpec((BLK, 1), lambda i: (i, 0)),
        out_shape=jax.ShapeDtypeStruct((BATCH, 1), jnp.float32),
    )(ug, mg, aux, W1u4, W1m4, War, b1, W2, b2, w3row, b3)
    return out


def kernel(user_ids, movie_ids, user_ages, movie_ratings,
           user_table, movie_table, W1, b1, W2, b2, W3, b3):
    uid = user_ids.astype(jnp.int32)
    mid = movie_ids.astype(jnp.int32)
    mt4 = _tc_regroup(movie_table.T, DM)
    mg = _sc_gather(mt4, (mid % DM).reshape(IDX_ROWS, CHUNK))
    ut4 = _tc_regroup(user_table.T, DU)
    ug = _sc_gather(ut4, (uid % DU).reshape(IDX_ROWS, CHUNK))
    aux = jnp.stack([(uid // DU).astype(jnp.float32),
                     (mid // DM).astype(jnp.float32),
                     user_ages.astype(jnp.float32),
                     movie_ratings.astype(jnp.float32)], axis=0)
    W1u4 = jnp.tile(W1[0:UD, :], (4, 1))
    W1m4 = jnp.tile(W1[UD:UD + MD, :], (4, 1))
    War = W1[UD + MD:UD + MD + 2, :]
    out = _mlp(ug, mg, aux, W1u4, W1m4, War, b1[None, :], W2, b2[None, :],
               W3.T, b3[None, :])
    return out[:, 0]


# speedup vs baseline: 2.1612x; 2.1612x over previous
import functools

import jax
import jax.numpy as jnp
from jax import lax
from jax.experimental import pallas as pl
from jax.experimental.pallas import tpu as pltpu
from jax.experimental.pallas import tpu_sc as plsc

BATCH = 16384
NUM_USERS = 1000000
NUM_MOVIES = 100000
UD = 32
MD = 32
H1 = 128
H2 = 64
GRP = 128

NUM_WORKERS = 32
CHUNK = 128
ROWS_PER_W = BATCH // NUM_WORKERS
CHUNKS_PER_W = ROWS_PER_W // CHUNK
IDX_ROWS = BATCH // CHUNK


def _sc_gather(t4, idx2d):
    mesh = plsc.VectorSubcoreMesh(core_axis_name="c", subcore_axis_name="s")

    @functools.partial(
        pl.kernel,
        mesh=mesh,
        compiler_params=pltpu.CompilerParams(use_tc_tiling_on_sc=False),
        out_type=jax.ShapeDtypeStruct((BATCH, GRP), jnp.float32),
        scratch_types=[
            pltpu.VMEM((CHUNKS_PER_W, CHUNK), jnp.int32),
            pltpu.VMEM((ROWS_PER_W, GRP), jnp.float32),
            pltpu.SemaphoreType.DMA,
        ],
    )
    def k(t_hbm, i_hbm, out_hbm, i_v, rows_v, sem):
        wid = lax.axis_index("s") * 2 + lax.axis_index("c")
        pltpu.sync_copy(i_hbm.at[pl.ds(wid * CHUNKS_PER_W, CHUNKS_PER_W)], i_v)
        copies = []
        for j in range(CHUNKS_PER_W):
            copies.append(pltpu.async_copy(
                t_hbm.at[i_v.at[j]],
                rows_v.at[pl.ds(j * CHUNK, CHUNK)], sem))
        for c in copies:
            c.wait()
        pltpu.sync_copy(rows_v, out_hbm.at[pl.ds(wid * ROWS_PER_W, ROWS_PER_W)])

    return k(t4, idx2d)


RSUB = 12544
DU = 250880
DM = 25088


def _regroup_body(x0_ref, x1_ref, x2_ref, x3_ref, o_ref):
    eye = jnp.eye(UD, dtype=jnp.float32)
    tr = lambda x: lax.dot_general(
        x, eye, (((0,), (0,)), ((), ())),
        preferred_element_type=jnp.float32)
    x = jnp.concatenate([x0_ref[...], x1_ref[...], x2_ref[...], x3_ref[...]],
                        axis=0)
    eye = jnp.eye(GRP, dtype=jnp.float32)
    o_ref[...] = lax.dot_general(x, eye, (((0,), (0,)), ((), ())),
                                 preferred_element_type=jnp.float32)


def _tc_regroup(tT, d):
    grid = d // RSUB
    specs = [
        pl.BlockSpec((UD, RSUB), lambda i, a=a, g=grid: (0, g * a + i))
        for a in range(4)
    ]
    return pl.pallas_call(
        _regroup_body,
        grid=(grid,),
        in_specs=specs,
        out_specs=pl.BlockSpec((RSUB, GRP), lambda i: (i, 0)),
        out_shape=jax.ShapeDtypeStruct((d, GRP), jnp.float32),
        compiler_params=pltpu.CompilerParams(
            dimension_semantics=("arbitrary",),
            vmem_limit_bytes=56 * 1024 * 1024),
    )(tT, tT, tT, tT)


BLK = 2048
GRID = BATCH // BLK


def _mlp_body(ug_ref, mg_ref, aux_ref, w1u_ref, w1m_ref, war_ref, b1_ref,
              w2_ref, b2_ref, w3_ref, b3_ref, out_ref):
    hp = jax.lax.Precision.DEFAULT
    f32 = jnp.float32
    aux = aux_ref[...]
    auxc = jnp.transpose(aux)
    uoff = auxc[:, 0:1].astype(jnp.int32)
    moff = auxc[:, 1:2].astype(jnp.int32)
    age = auxc[:, 2:3]
    rat = auxc[:, 3:4]
    colgrp = lax.broadcasted_iota(jnp.int32, (BLK, GRP), 1) // UD
    um = jnp.where(colgrp == uoff, ug_ref[...], 0.0)
    mm = jnp.where(colgrp == moff, mg_ref[...], 0.0)
    h = jnp.dot(um, w1u_ref[...], precision=hp, preferred_element_type=f32)
    h = h + jnp.dot(mm, w1m_ref[...], precision=hp, preferred_element_type=f32)
    war = war_ref[...]
    h = h + age * war[0:1, :] + rat * war[1:2, :]
    h = jnp.maximum(h + b1_ref[...], 0.0)
    h = jnp.dot(h, w2_ref[...], precision=hp, preferred_element_type=f32)
    h = jnp.maximum(h + b2_ref[...], 0.0)
    logit = jnp.sum(h * w3_ref[...], axis=1, keepdims=True) + b3_ref[...]
    out_ref[...] = jax.nn.sigmoid(logit)


def _mlp(ug, mg, aux, W1u4, W1m4, War, b1, W2, b2, w3row, b3):
    full = lambda i: (0, 0)
    out = pl.pallas_call(
        _mlp_body,
        grid=(GRID,),
        in_specs=[
            pl.BlockSpec((BLK, GRP), lambda i: (i, 0)),
            pl.BlockSpec((BLK, GRP), lambda i: (i, 0)),
            pl.BlockSpec((4, BLK), lambda i: (0, i)),
            pl.BlockSpec((GRP, H1), full),
            pl.BlockSpec((GRP, H1), full),
            pl.BlockSpec((2, H1), full),
            pl.BlockSpec((1, H1), full),
            pl.BlockSpec((H1, H2), full),
            pl.BlockSpec((1, H2), full),
            pl.BlockSpec((1, H2), full),
            pl.BlockSpec((1, 1), full),
        ],
        out_specs=pl.BlockSpec((BLK, 1), lambda i: (i, 0)),
        out_shape=jax.ShapeDtypeStruct((BATCH, 1), jnp.float32),
    )(ug, mg, aux, W1u4, W1m4, War, b1, W2, b2, w3row, b3)
    return out


def kernel(user_ids, movie_ids, user_ages, movie_ratings,
           user_table, movie_table, W1, b1, W2, b2, W3, b3):
    uid = user_ids.astype(jnp.int32)
    mid = movie_ids.astype(jnp.int32)
    mt4 = _tc_regroup(movie_table.T, DM)
    mg = _sc_gather(mt4, (mid % DM).reshape(IDX_ROWS, CHUNK))
    ut4 = _tc_regroup(user_table.T, DU)
    ug = _sc_gather(ut4, (uid % DU).reshape(IDX_ROWS, CHUNK))
    aux = jnp.stack([(uid // DU).astype(jnp.float32),
                     (mid // DM).astype(jnp.float32),
                     user_ages.astype(jnp.float32),
                     movie_ratings.astype(jnp.float32)], axis=0)
    W1u4 = jnp.tile(W1[0:UD, :], (4, 1))
    W1m4 = jnp.tile(W1[UD:UD + MD, :], (4, 1))
    War = W1[UD + MD:UD + MD + 2, :]
    out = _mlp(ug, mg, aux, W1u4, W1m4, War, b1[None, :], W2, b2[None, :],
               W3.T, b3[None, :])
    return out[:, 0]
